# Initial kernel scaffold; baseline (speedup 1.0000x reference)
import jax
import jax.numpy as jnp
from jax.experimental import pallas as pl


def kernel(x, edge_index, edge_attr, batch_idx, emb, W1, a_src1, a_dst1, b1, W2, a_src2, a_dst2, b2, W3, a_src3, a_dst3, b3, lin1_w, lin1_b, lin2_w, lin2_b):
    raise NotImplementedError("write your pallas kernel here")


# speedup vs baseline: 39.0328x; 39.0328x over previous
import functools

import jax
import jax.numpy as jnp
from jax import lax
from jax.experimental import pallas as pl
from jax.experimental.pallas import tpu as pltpu
from jax.experimental.pallas import tpu_sc as plsc

N = 10000
E = 320000
HID = 128
HEADS = 2
G = 64

NC = 2
NS = 16
NPAD = 10240

EPW = E // NS
CHK = 80
NCHUNK = EPW // CHK
RPT = NPAD // NS

_MESH = plsc.VectorSubcoreMesh(
    core_axis_name="c", subcore_axis_name="s", num_cores=NC, num_subcores=NS)


def _embed_body(xit, emb, h0, idxb, gcb, acc, sem):
  c = lax.axis_index("c")
  s = lax.axis_index("s")
  wid = s * NC + c
  nch = NPAD // 128
  for k in range(3):
    ch = wid + k * 32

    @pl.when(ch < nch)
    def _():
      r0 = ch * 128
      for j in range(9):
        pltpu.sync_copy(xit.at[pl.ds(j * NPAD + r0, 128)], idxb)
        pltpu.async_copy(emb.at[idxb], gcb, sem).wait()
        if j == 0:
          def cp(r, _):
            for q in range(8):
              acc[r, pl.ds(q * 16, 16)] = gcb[r, pl.ds(q * 16, 16)]
            return 0
          lax.fori_loop(0, 128, cp, 0)
        else:
          def ad(r, _):
            for q in range(8):
              acc[r, pl.ds(q * 16, 16)] = (
                  acc[r, pl.ds(q * 16, 16)] + gcb[r, pl.ds(q * 16, 16)])
            return 0
          lax.fori_loop(0, 128, ad, 0)
      pltpu.sync_copy(acc, h0.at[pl.ds(r0, 128)])


_SC_PARAMS = pltpu.CompilerParams(needs_layout_passes=False)

_embed_call = pl.kernel(
    _embed_body,
    out_type=jax.ShapeDtypeStruct((NPAD, HID), jnp.float32),
    mesh=_MESH,
    compiler_params=_SC_PARAMS,
    scratch_types=[
        pltpu.VMEM((128,), jnp.int32),
        pltpu.VMEM((128, HID), jnp.float32),
        pltpu.VMEM((128, HID), jnp.float32),
        pltpu.SemaphoreType.DMA,
    ],
)


def _edge_body(hw, alpha, src, dst, bmat, hout,
               alpha_v, srcb, dstb, gidx, sbuf, gbuf, denstage, bbuf,
               acc_sp, den_sp, sem):
  c = lax.axis_index("c")
  s = lax.axis_index("s")
  cnp = c * NPAD
  zero16 = jnp.zeros((16,), jnp.float32)

  def zrow(r, _):
    for q in range(8):
      gbuf[r, pl.ds(q * 16, 16)] = zero16
    return 0
  lax.fori_loop(0, CHK, zrow, 0)

  def zden(i, _):
    denstage[pl.ds(i * 16, 16)] = zero16
    return 0
  lax.fori_loop(0, RPT // 16, zden, 0)

  r0 = s * RPT
  for t in range(RPT // CHK):
    pltpu.sync_copy(gbuf, acc_sp.at[pl.ds(r0 + t * CHK, CHK)])
  pltpu.sync_copy(denstage, den_sp.at[pl.ds(r0, RPT)])

  pltpu.sync_copy(alpha.at[pl.ds(c * 2 * NPAD, 2 * NPAD)], alpha_v)
  pltpu.sync_copy(bmat.at[pl.ds(c, 1)], bbuf)
  plsc.subcore_barrier()

  e_base = s * EPW

  def chunk_body(ch, _):
    e0 = e_base + ch * CHK
    pltpu.sync_copy(src.at[pl.ds(e0, CHK)], srcb)
    pltpu.sync_copy(dst.at[pl.ds(e0, CHK)], dstb)
    for k in range(CHK // 16):
      gidx[pl.ds(k * 16, 16)] = srcb[pl.ds(k * 16, 16)] + cnp
    cp = pltpu.async_copy(hw.at[gidx], gbuf, sem)
    for k in range(CHK // 16):
      sv = srcb[pl.ds(k * 16, 16)]
      dv = dstb[pl.ds(k * 16, 16)]
      a_s = plsc.load_gather(alpha_v, [sv * 2])
      a_d = plsc.load_gather(alpha_v, [dv * 2 + 1])
      ev = a_s + a_d
      ev = jnp.where(ev > 0, ev, 0.2 * ev)
      sbuf[pl.ds(k * 16, 16)] = jnp.exp(ev)
    pltpu.sync_copy(sbuf, den_sp.at[dstb], add=True)
    cp.wait()

    def rscale(j, _):
      sj = plsc.load_gather(sbuf, [jnp.zeros((16,), jnp.int32) + j])
      for q in range(8):
        gbuf[j, pl.ds(q * 16, 16)] = gbuf[j, pl.ds(q * 16, 16)] * sj
      return 0
    lax.fori_loop(0, CHK, rscale, 0)
    pltpu.sync_copy(gbuf, acc_sp.at[dstb], add=True)
    return 0
  lax.fori_loop(0, NCHUNK, chunk_body, 0)

  plsc.subcore_barrier()
  pltpu.sync_copy(den_sp.at[pl.ds(r0, RPT)], denstage)

  for t in range(RPT // CHK):
    rc = r0 + t * CHK
    pltpu.sync_copy(acc_sp.at[pl.ds(rc, CHK)], gbuf)

    def erow(j, _):
      dj = plsc.load_gather(
          denstage, [jnp.zeros((16,), jnp.int32) + (t * CHK + j)])
      rin = 1.0 / (dj + 1e-16)
      for q in range(8):
        v = gbuf[j, pl.ds(q * 16, 16)] * rin + bbuf[0, pl.ds(q * 16, 16)]
        gbuf[j, pl.ds(q * 16, 16)] = jnp.where(v > 0, v, jnp.exp(v) - 1.0)
      return 0
    lax.fori_loop(0, CHK, erow, 0)
    pltpu.sync_copy(gbuf, hout.at[pl.ds(cnp + rc, CHK)])


_edge_call = pl.kernel(
    _edge_body,
    out_type=jax.ShapeDtypeStruct((NC * NPAD, HID), jnp.float32),
    mesh=_MESH,
    compiler_params=_SC_PARAMS,
    scratch_types=[
        pltpu.VMEM((NPAD * 2,), jnp.float32),
        pltpu.VMEM((CHK,), jnp.int32),
        pltpu.VMEM((CHK,), jnp.int32),
        pltpu.VMEM((CHK,), jnp.int32),
        pltpu.VMEM((CHK,), jnp.float32),
        pltpu.VMEM((CHK, HID), jnp.float32),
        pltpu.VMEM((RPT,), jnp.float32),
        pltpu.VMEM((1, HID), jnp.float32),
        pltpu.VMEM_SHARED((NPAD, HID), jnp.float32),
        pltpu.VMEM_SHARED((NPAD,), jnp.float32),
        pltpu.SemaphoreType.DMA,
    ],
)


def _make_dense(cin_heads):
  R = 640
  cin = HEADS * HID if cin_heads else HID

  def body(h_ref, w_ref, amat_ref, hw_ref, alpha_ref):
    if cin_heads:
      hb = jnp.concatenate([h_ref[0], h_ref[1]], axis=1)
    else:
      hb = h_ref[...]
    hw = jnp.dot(hb, w_ref[...], preferred_element_type=jnp.float32)
    hw_ref[0] = hw[:, :HID]
    hw_ref[1] = hw[:, HID:]
    alpha4 = jnp.dot(hw, amat_ref[...], preferred_element_type=jnp.float32)
    alpha_ref[0] = alpha4[:, 0:2]
    alpha_ref[1] = alpha4[:, 2:4]

  if cin_heads:
    h_spec = pl.BlockSpec((2, R, HID), lambda i: (0, i, 0))
  else:
    h_spec = pl.BlockSpec((R, HID), lambda i: (i, 0))
  return pl.pallas_call(
      body,
      grid=(NPAD // R,),
      in_specs=[
          h_spec,
          pl.BlockSpec((cin, HEADS * HID), lambda i: (0, 0)),
          pl.BlockSpec((HEADS * HID, 4), lambda i: (0, 0)),
      ],
      out_specs=[
          pl.BlockSpec((2, R, HID), lambda i: (0, i, 0)),
          pl.BlockSpec((2, R, 2), lambda i: (0, i, 0)),
      ],
      out_shape=[
          jax.ShapeDtypeStruct((2, NPAD, HID), jnp.float32),
          jax.ShapeDtypeStruct((2, NPAD, 2), jnp.float32),
      ],
  )


_dense1 = _make_dense(False)
_dense2 = _make_dense(True)


def _pool_body(h_ref, bidx_ref, l1w_ref, l1b_ref, l2w_ref, l2b_ref,
               out_ref, pool_acc, cnt_acc):
  i = pl.program_id(0)

  @pl.when(i == 0)
  def _():
    pool_acc[...] = jnp.zeros_like(pool_acc)
    cnt_acc[...] = jnp.zeros_like(cnt_acc)

  hb = jnp.concatenate([h_ref[0], h_ref[1]], axis=1)
  bidx = bidx_ref[...]
  ids = lax.broadcasted_iota(jnp.int32, (1, G), 1)
  onehot = (bidx == ids).astype(jnp.float32)
  pool_acc[...] += lax.dot_general(
      onehot, hb, (((0,), (0,)), ((), ())), preferred_element_type=jnp.float32)
  cnt_acc[...] += lax.dot_general(
      onehot, jnp.ones((hb.shape[0], 8), jnp.float32),
      (((0,), (0,)), ((), ())), preferred_element_type=jnp.float32)

  @pl.when(i == pl.num_programs(0) - 1)
  def _():
    cnt = jnp.maximum(cnt_acc[:, 0:1], 1.0)
    gm = pool_acc[...] / cnt
    gm = jnp.maximum(
        jnp.dot(gm, l1w_ref[...], preferred_element_type=jnp.float32)
        + l1b_ref[...], 0.0)
    out_ref[...] = (
        jnp.dot(gm, l2w_ref[...], preferred_element_type=jnp.float32)
        + l2b_ref[...])


_pool_call = pl.pallas_call(
    _pool_body,
    grid=(NPAD // 640,),
    in_specs=[
        pl.BlockSpec((2, 640, HID), lambda i: (0, i, 0)),
        pl.BlockSpec((640, 1), lambda i: (i, 0)),
        pl.BlockSpec((HEADS * HID, HID), lambda i: (0, 0)),
        pl.BlockSpec((1, HID), lambda i: (0, 0)),
        pl.BlockSpec((HID, 1), lambda i: (0, 0)),
        pl.BlockSpec((1, 1), lambda i: (0, 0)),
    ],
    out_specs=pl.BlockSpec((G, 1), lambda i: (0, 0)),
    out_shape=jax.ShapeDtypeStruct((G, 1), jnp.float32),
    scratch_shapes=[
        pltpu.VMEM((G, HEADS * HID), jnp.float32),
        pltpu.VMEM((G, 8), jnp.float32),
    ],
)


def _amat(a_src, a_dst):
  z = jnp.zeros((HID,), jnp.float32)
  cols = [
      jnp.concatenate([a_src[0], z]),
      jnp.concatenate([a_dst[0], z]),
      jnp.concatenate([z, a_src[1]]),
      jnp.concatenate([z, a_dst[1]]),
  ]
  return jnp.stack(cols, axis=1)


@jax.jit
def kernel(x, edge_index, edge_attr, batch_idx, emb,
           W1, a_src1, a_dst1, b1,
           W2, a_src2, a_dst2, b2,
           W3, a_src3, a_dst3, b3,
           lin1_w, lin1_b, lin2_w, lin2_b):
  del edge_attr
  x = x.astype(jnp.int32)
  feat_off = 1 + jnp.arange(0, 9 * 512, 512, dtype=jnp.int32)
  xi = (x + feat_off[None, :]).T
  xit = jnp.zeros((9, NPAD), jnp.int32).at[:, :N].set(xi).reshape(-1)
  src = edge_index[0].astype(jnp.int32)
  dst = edge_index[1].astype(jnp.int32)
  bidx = jnp.full((NPAD, 1), G, jnp.int32).at[:N, 0].set(
      batch_idx.astype(jnp.int32))

  h = _embed_call(xit, emb)
  layers = [
      (W1, a_src1, a_dst1, b1, _dense1),
      (W2, a_src2, a_dst2, b2, _dense2),
      (W3, a_src3, a_dst3, b3, _dense2),
  ]
  for W, a_s, a_d, b, dense in layers:
    hw, alpha = dense(h, W, _amat(a_s, a_d))
    hf = _edge_call(hw.reshape(NC * NPAD, HID), alpha.reshape(-1), src, dst,
                    b.reshape(HEADS, HID))
    h = hf.reshape(NC, NPAD, HID)

  return _pool_call(h, bidx, lin1_w, lin1_b.reshape(1, HID),
                    lin2_w, lin2_b.reshape(1, 1))
